# Initial kernel scaffold; baseline (speedup 1.0000x reference)
import jax
import jax.numpy as jnp
from jax.experimental import pallas as pl


def kernel(hidden_states, route, W1, b1, W2, b2, gamma, beta):
    raise NotImplementedError("write your pallas kernel here")


# speedup vs baseline: 1.7209x; 1.7209x over previous
import functools

import jax
import jax.numpy as jnp
from jax import lax
from jax.experimental import pallas as pl
from jax.experimental.pallas import tpu as pltpu
from jax.experimental.pallas import tpu_sc as plsc

E = 16
D = 512
F = 1024
TM = 128
EPS = 1e-12


def _sc_row_gather(table, idx):
    n, d = idx.shape[0], table.shape[1]
    info = plsc.get_sparse_core_info()
    nw = info.num_cores * info.num_subcores
    per_w = n // nw
    ch = min(per_w, 64)
    n_ch = per_w // ch
    mesh = plsc.VectorSubcoreMesh(core_axis_name="c", subcore_axis_name="s")

    @functools.partial(
        pl.kernel,
        mesh=mesh,
        out_type=jax.ShapeDtypeStruct((n, d), table.dtype),
        scratch_types=[
            pltpu.VMEM((ch,), jnp.int32),
            pltpu.VMEM((ch, d), table.dtype),
            pltpu.SemaphoreType.DMA,
        ],
    )
    def gather_k(table_hbm, idx_hbm, out_hbm, idx_v, rows_v, sem):
        wid = lax.axis_index("s") * info.num_cores + lax.axis_index("c")
        base = wid * per_w

        def body(i, carry):
            off = base + i * ch
            pltpu.sync_copy(idx_hbm.at[pl.ds(off, ch)], idx_v)
            pltpu.async_copy(table_hbm.at[idx_v], rows_v, sem).wait()
            pltpu.sync_copy(rows_v, out_hbm.at[pl.ds(off, ch)])
            return carry

        lax.fori_loop(0, n_ch, body, 0)

    return gather_k(table, idx)


def _mlp_body(eids_ref, x_ref, w1_ref, b1_ref, w2_ref, b2_ref, gm_ref, bt_ref,
              o_ref):
    x = x_ref[...]
    h = lax.dot_general(x, w1_ref[0], (((1,), (1,)), ((), ())),
                        preferred_element_type=jnp.float32)
    h = jax.nn.gelu(h + b1_ref[0])
    y = lax.dot_general(h, w2_ref[0], (((1,), (1,)), ((), ())),
                        preferred_element_type=jnp.float32)
    z = y + b2_ref[0] + x
    mu = jnp.mean(z, axis=1, keepdims=True)
    zc = z - mu
    var = jnp.mean(zc * zc, axis=1, keepdims=True)
    zn = zc * lax.rsqrt(var + EPS)
    o_ref[...] = zn * gm_ref[...] + bt_ref[...]


def kernel(hidden_states, route, W1, b1, W2, b2, gamma, beta):
    b, s, _ = hidden_states.shape
    t = b * s
    t_pad = t + E * TM
    g = t_pad // TM

    x = hidden_states.reshape(t, D)
    r = route.astype(jnp.int32)

    oh = (r[:, None] == jnp.arange(E, dtype=jnp.int32)[None, :]).astype(
        jnp.int32)
    occ = jnp.cumsum(oh, axis=0) - oh
    rank = jnp.take_along_axis(occ, r[:, None], axis=1)[:, 0]
    counts = jnp.sum(oh, axis=0)
    padded = ((counts + TM - 1) // TM) * TM
    po = jnp.cumsum(padded) - padded
    dest = po[r] + rank
    inv = jnp.zeros((t_pad,), jnp.int32).at[dest].set(
        jnp.arange(t, dtype=jnp.int32))
    tile_start = po // TM
    eids = (jnp.searchsorted(tile_start,
                             jnp.arange(g, dtype=jnp.int32),
                             side="right") - 1).astype(jnp.int32)

    x_pad = _sc_row_gather(x, inv)

    grid_spec = pltpu.PrefetchScalarGridSpec(
        num_scalar_prefetch=1,
        grid=(g,),
        in_specs=[
            pl.BlockSpec((TM, D), lambda i, e: (i, 0)),
            pl.BlockSpec((1, F, D), lambda i, e: (e[i], 0, 0)),
            pl.BlockSpec((1, 1, F), lambda i, e: (e[i], 0, 0)),
            pl.BlockSpec((1, D, F), lambda i, e: (e[i], 0, 0)),
            pl.BlockSpec((1, 1, D), lambda i, e: (e[i], 0, 0)),
            pl.BlockSpec((1, D), lambda i, e: (0, 0)),
            pl.BlockSpec((1, D), lambda i, e: (0, 0)),
        ],
        out_specs=pl.BlockSpec((TM, D), lambda i, e: (i, 0)),
    )
    out_pad = pl.pallas_call(
        _mlp_body,
        grid_spec=grid_spec,
        out_shape=jax.ShapeDtypeStruct((t_pad, D), jnp.float32),
    )(eids, x_pad, W1, b1.reshape(E, 1, F), W2, b2.reshape(E, 1, D),
      gamma.reshape(1, D), beta.reshape(1, D))

    y = _sc_row_gather(out_pad, dest)
    return y.reshape(b, s, D)
